# Initial kernel scaffold; baseline (speedup 1.0000x reference)
import jax
import jax.numpy as jnp
from jax.experimental import pallas as pl


def kernel(x_hat, lam_hat, A_rows, A_cols, A_vals, b_pad, c_pad, b_mask, c_mask):
    raise NotImplementedError("write your pallas kernel here")


# speedup vs baseline: 230.9073x; 230.9073x over previous
import functools

import jax
import jax.numpy as jnp
from jax import lax
from jax.experimental import pallas as pl
from jax.experimental.pallas import tpu as pltpu
from jax.experimental.pallas import tpu_sc as plsc

B = 4
M = 16384
N = 16384
NNZ = 262144
W_PRIMAL, W_DUAL, W_STAT, W_COMP = 0.1, 0.1, 0.6, 0.2

NC = 2
NS = 16
L = 16
W_PER_P = 8
NNZ_W = NNZ // W_PER_P
CHUNK = 8192
NCHUNKS = NNZ_W // CHUNK
RED = M // W_PER_P

_mesh = plsc.VectorSubcoreMesh(core_axis_name="c", subcore_axis_name="s",
                               num_cores=NC, num_subcores=NS)


_KERNEL_KWARGS = dict(
    out_type=jax.ShapeDtypeStruct((NC, NS, 4, L), jnp.float32),
    mesh=_mesh,
    compiler_params=pltpu.CompilerParams(needs_layout_passes=False),
    scratch_types=[
        pltpu.VMEM((N,), jnp.float32),
        pltpu.VMEM((M,), jnp.float32),
        pltpu.VMEM((M,), jnp.float32),
        pltpu.VMEM((N,), jnp.float32),
        pltpu.VMEM((CHUNK,), jnp.int32),
        pltpu.VMEM((CHUNK,), jnp.int32),
        pltpu.VMEM((CHUNK,), jnp.float32),
        pltpu.VMEM((4, L), jnp.float32),
        pltpu.VMEM_SHARED((NS, M), jnp.float32),
        pltpu.VMEM_SHARED((NS, N), jnp.float32),
    ],
)


def _kkt_sc_body(x_hbm, lam_hbm, rows_hbm, cols_hbm, vals_hbm, b_hbm, c_hbm,
            out_hbm, x_v, lam_v, ax_v, at_v, rows_v, cols_v, vals_v,
            out_v, sh_ax, sh_at):
    c = lax.axis_index("c")
    s = lax.axis_index("s")
    p = c * (B // NC) + s // W_PER_P
    sl = s % W_PER_P

    pltpu.sync_copy(x_hbm.at[pl.ds(p * N, N)], x_v)
    pltpu.sync_copy(lam_hbm.at[pl.ds(p * M, M)], lam_v)

    zeros = jnp.zeros((L,), jnp.float32)

    def zero_body(i, _):
        ax_v[pl.ds(i * L, L)] = zeros
        at_v[pl.ds(i * L, L)] = zeros
        return 0

    lax.fori_loop(0, M // L, zero_body, 0)

    nnz_base = p * NNZ + sl * NNZ_W
    for k in range(NCHUNKS):
        off = nnz_base + k * CHUNK
        pltpu.sync_copy(rows_hbm.at[pl.ds(off, CHUNK)], rows_v)
        pltpu.sync_copy(cols_hbm.at[pl.ds(off, CHUNK)], cols_v)
        pltpu.sync_copy(vals_hbm.at[pl.ds(off, CHUNK)], vals_v)

        def spmm_body(j, _):
            rows = rows_v[pl.ds(j * L, L)]
            cols = cols_v[pl.ds(j * L, L)]
            vals = vals_v[pl.ds(j * L, L)]
            gx = plsc.load_gather(x_v, [cols])
            plsc.addupdate_scatter(ax_v, [rows], vals * gx)
            gl = plsc.load_gather(lam_v, [rows])
            plsc.addupdate_scatter(at_v, [cols], vals * gl)
            return 0

        lax.fori_loop(0, CHUNK // L, spmm_body, 0)

    pltpu.sync_copy(ax_v, sh_ax.at[s])
    pltpu.sync_copy(at_v, sh_at.at[s])
    plsc.subcore_barrier()

    row0 = (s // W_PER_P) * W_PER_P
    mbase = sl * RED
    for j in range(W_PER_P):
        pltpu.sync_copy(sh_ax.at[row0 + j, pl.ds(mbase, RED)],
                        ax_v.at[pl.ds(j * RED, RED)])
        pltpu.sync_copy(sh_at.at[row0 + j, pl.ds(mbase, RED)],
                        at_v.at[pl.ds(j * RED, RED)])
    pltpu.sync_copy(b_hbm.at[pl.ds(p * M + mbase, RED)], vals_v.at[pl.ds(0, RED)])
    pltpu.sync_copy(c_hbm.at[pl.ds(p * N + mbase, RED)], vals_v.at[pl.ds(RED, RED)])

    def red_body(i, carry):
        vp, vd, vs_, vc = carry
        ax = ax_v[pl.ds(i * L, L)]
        at = at_v[pl.ds(i * L, L)]
        for j in range(1, W_PER_P):
            ax = ax + ax_v[pl.ds(j * RED + i * L, L)]
            at = at + at_v[pl.ds(j * RED + i * L, L)]
        amb = ax - vals_v[pl.ds(i * L, L)]
        lamv = lam_v[pl.ds(mbase + i * L, L)]
        rp = jnp.maximum(amb, 0.0)
        rd = jnp.maximum(-lamv, 0.0)
        st = at + vals_v[pl.ds(RED + i * L, L)]
        cm = lamv * amb
        return (vp + rp * rp, vd + rd * rd, vs_ + st * st, vc + cm * cm)

    vp, vd, vs_, vc = lax.fori_loop(
        0, RED // L, red_body, (zeros, zeros, zeros, zeros))

    out_v[0] = vp
    out_v[1] = vd
    out_v[2] = vs_
    out_v[3] = vc
    pltpu.sync_copy(out_v, out_hbm.at[c, s])


_kkt_sc = pl.kernel(_kkt_sc_body, **_KERNEL_KWARGS)


def kernel(x_hat, lam_hat, A_rows, A_cols, A_vals, b_pad, c_pad, b_mask, c_mask):
    del b_mask, c_mask
    parts = _kkt_sc(
        x_hat.astype(jnp.float32),
        lam_hat.astype(jnp.float32),
        A_rows.reshape(-1),
        A_cols.reshape(-1),
        A_vals.reshape(-1).astype(jnp.float32),
        b_pad.reshape(-1).astype(jnp.float32),
        c_pad.reshape(-1).astype(jnp.float32),
    )
    sums = jnp.sum(parts, axis=(0, 1, 3))
    inv_B = 1.0 / B
    return (W_PRIMAL * (sums[0] / M) * inv_B
            + W_DUAL * (sums[1] / M) * inv_B
            + W_STAT * (sums[2] / N) * inv_B
            + W_COMP * (sums[3] / M) * inv_B)


# speedup vs baseline: 314.2935x; 1.3611x over previous
import jax
import jax.numpy as jnp
from jax import lax
from jax.experimental import pallas as pl
from jax.experimental.pallas import tpu as pltpu
from jax.experimental.pallas import tpu_sc as plsc

B = 4
M = 16384
N = 16384
NNZ = 262144
W_PRIMAL, W_DUAL, W_STAT, W_COMP = 0.1, 0.1, 0.6, 0.2

NC = 2
NS = 16
L = 16
W_PER_P = 8
NNZ_W = NNZ // W_PER_P
CHUNK = 8192
NCHUNKS = NNZ_W // CHUNK
RED = M // W_PER_P

_mesh = plsc.VectorSubcoreMesh(core_axis_name="c", subcore_axis_name="s",
                               num_cores=NC, num_subcores=NS)

_KERNEL_KWARGS = dict(
    out_type=jax.ShapeDtypeStruct((NC, NS, 4, L), jnp.float32),
    mesh=_mesh,
    compiler_params=pltpu.CompilerParams(needs_layout_passes=False),
    scratch_types=[
        pltpu.VMEM((N,), jnp.float32),
        pltpu.VMEM((M,), jnp.float32),
        pltpu.VMEM((M,), jnp.float32),
        pltpu.VMEM((N,), jnp.float32),
        pltpu.VMEM((CHUNK,), jnp.int32),
        pltpu.VMEM((CHUNK,), jnp.int32),
        pltpu.VMEM((CHUNK,), jnp.float32),
        pltpu.VMEM((4, L), jnp.float32),
        pltpu.VMEM_SHARED((NS, M), jnp.float32),
        pltpu.VMEM_SHARED((NS, N), jnp.float32),
    ],
)


def _kkt_sc_body(x_hbm, lam_hbm, rows_hbm, cols_hbm, vals_hbm, b_hbm, c_hbm,
                 out_hbm, x_v, lam_v, ax_v, at_v, rows_v, cols_v, vals_v,
                 out_v, sh_ax, sh_at):
    c = lax.axis_index("c")
    s = lax.axis_index("s")
    p = c * (B // NC) + s // W_PER_P
    sl = s % W_PER_P

    pltpu.sync_copy(x_hbm.at[pl.ds(p * N, N)], x_v)
    pltpu.sync_copy(lam_hbm.at[pl.ds(p * M, M)], lam_v)

    zeros = jnp.zeros((L,), jnp.float32)

    @plsc.parallel_loop(0, M // L, unroll=8)
    def _zero(i):
        ax_v[pl.ds(i * L, L)] = zeros
        at_v[pl.ds(i * L, L)] = zeros

    nnz_base = p * NNZ + sl * NNZ_W
    for k in range(NCHUNKS):
        off = nnz_base + k * CHUNK
        pltpu.sync_copy(rows_hbm.at[pl.ds(off, CHUNK)], rows_v)
        pltpu.sync_copy(cols_hbm.at[pl.ds(off, CHUNK)], cols_v)
        pltpu.sync_copy(vals_hbm.at[pl.ds(off, CHUNK)], vals_v)

        @plsc.parallel_loop(0, CHUNK // L, unroll=4)
        def _spmm(j):
            rows = rows_v[pl.ds(j * L, L)]
            cols = cols_v[pl.ds(j * L, L)]
            vals = vals_v[pl.ds(j * L, L)]
            gx = plsc.load_gather(x_v, [cols])
            plsc.addupdate_scatter(ax_v, [rows], vals * gx)
            gl = plsc.load_gather(lam_v, [rows])
            plsc.addupdate_scatter(at_v, [cols], vals * gl)

    pltpu.sync_copy(ax_v, sh_ax.at[s])
    pltpu.sync_copy(at_v, sh_at.at[s])
    plsc.subcore_barrier()

    row0 = (s // W_PER_P) * W_PER_P
    mbase = sl * RED
    for j in range(W_PER_P):
        pltpu.sync_copy(sh_ax.at[row0 + j, pl.ds(mbase, RED)],
                        ax_v.at[pl.ds(j * RED, RED)])
        pltpu.sync_copy(sh_at.at[row0 + j, pl.ds(mbase, RED)],
                        at_v.at[pl.ds(j * RED, RED)])
    pltpu.sync_copy(b_hbm.at[pl.ds(p * M + mbase, RED)], vals_v.at[pl.ds(0, RED)])
    pltpu.sync_copy(c_hbm.at[pl.ds(p * N + mbase, RED)], vals_v.at[pl.ds(RED, RED)])

    carry0 = (zeros, zeros, zeros, zeros)

    @plsc.parallel_loop(0, RED // L, unroll=2, carry=carry0)
    def _red(i, carry):
        vp, vd, vs_, vc = carry
        ax = ax_v[pl.ds(i * L, L)]
        at = at_v[pl.ds(i * L, L)]
        for j in range(1, W_PER_P):
            ax = ax + ax_v[pl.ds(j * RED + i * L, L)]
            at = at + at_v[pl.ds(j * RED + i * L, L)]
        amb = ax - vals_v[pl.ds(i * L, L)]
        lamv = lam_v[pl.ds(mbase + i * L, L)]
        rp = jnp.maximum(amb, 0.0)
        rd = jnp.maximum(-lamv, 0.0)
        st = at + vals_v[pl.ds(RED + i * L, L)]
        cm = lamv * amb
        return (vp + rp * rp, vd + rd * rd, vs_ + st * st, vc + cm * cm)

    vp, vd, vs_, vc = _red
    out_v[0] = vp
    out_v[1] = vd
    out_v[2] = vs_
    out_v[3] = vc
    pltpu.sync_copy(out_v, out_hbm.at[c, s])


_kkt_sc = pl.kernel(_kkt_sc_body, **_KERNEL_KWARGS)


def kernel(x_hat, lam_hat, A_rows, A_cols, A_vals, b_pad, c_pad, b_mask, c_mask):
    del b_mask, c_mask
    parts = _kkt_sc(
        x_hat.astype(jnp.float32),
        lam_hat.astype(jnp.float32),
        A_rows.reshape(-1),
        A_cols.reshape(-1),
        A_vals.reshape(-1).astype(jnp.float32),
        b_pad.reshape(-1).astype(jnp.float32),
        c_pad.reshape(-1).astype(jnp.float32),
    )
    sums = jnp.sum(parts, axis=(0, 1, 3))
    inv_B = 1.0 / B
    return (W_PRIMAL * (sums[0] / M) * inv_B
            + W_DUAL * (sums[1] / M) * inv_B
            + W_STAT * (sums[2] / N) * inv_B
            + W_COMP * (sums[3] / M) * inv_B)


# speedup vs baseline: 365.5505x; 1.1631x over previous
import jax
import jax.numpy as jnp
from jax import lax
from jax.experimental import pallas as pl
from jax.experimental.pallas import tpu as pltpu
from jax.experimental.pallas import tpu_sc as plsc

B = 4
M = 16384
N = 16384
NNZ = 262144
W_PRIMAL, W_DUAL, W_STAT, W_COMP = 0.1, 0.1, 0.6, 0.2

NC = 2
NS = 16
L = 16
W_PER_P = 8
NNZ_W = NNZ // W_PER_P
CHUNK = 4096
NCHUNKS = NNZ_W // CHUNK
RED = M // W_PER_P

_mesh = plsc.VectorSubcoreMesh(core_axis_name="c", subcore_axis_name="s",
                               num_cores=NC, num_subcores=NS)

_KERNEL_KWARGS = dict(
    out_type=jax.ShapeDtypeStruct((NC, NS, 4, L), jnp.float32),
    mesh=_mesh,
    compiler_params=pltpu.CompilerParams(needs_layout_passes=False),
    scratch_types=[
        pltpu.VMEM((N,), jnp.float32),
        pltpu.VMEM((M,), jnp.float32),
        pltpu.VMEM((M,), jnp.float32),
        pltpu.VMEM((N,), jnp.float32),
        pltpu.VMEM((CHUNK,), jnp.int32),
        pltpu.VMEM((CHUNK,), jnp.int32),
        pltpu.VMEM((CHUNK,), jnp.float32),
        pltpu.VMEM((CHUNK,), jnp.int32),
        pltpu.VMEM((CHUNK,), jnp.int32),
        pltpu.VMEM((CHUNK,), jnp.float32),
        pltpu.VMEM((4, L), jnp.float32),
        pltpu.VMEM_SHARED((NS, M), jnp.float32),
        pltpu.VMEM_SHARED((NS, N), jnp.float32),
        pltpu.SemaphoreType.DMA,
        pltpu.SemaphoreType.DMA,
        pltpu.SemaphoreType.DMA,
        pltpu.SemaphoreType.DMA,
        pltpu.SemaphoreType.DMA,
        pltpu.SemaphoreType.DMA,
    ],
)


def _kkt_sc_body(x_hbm, lam_hbm, rows_hbm, cols_hbm, vals_hbm, b_hbm, c_hbm,
                 out_hbm, x_v, lam_v, ax_v, at_v,
                 rows0, cols0, vals0, rows1, cols1, vals1,
                 out_v, sh_ax, sh_at,
                 sem_r0, sem_c0, sem_v0, sem_r1, sem_c1, sem_v1):
    c = lax.axis_index("c")
    s = lax.axis_index("s")
    p = c * (B // NC) + s // W_PER_P
    sl = s % W_PER_P

    pltpu.sync_copy(x_hbm.at[pl.ds(p * N, N)], x_v)
    pltpu.sync_copy(lam_hbm.at[pl.ds(p * M, M)], lam_v)

    zeros = jnp.zeros((L,), jnp.float32)

    @plsc.parallel_loop(0, M // L, unroll=8)
    def _zero(i):
        ax_v[pl.ds(i * L, L)] = zeros
        at_v[pl.ds(i * L, L)] = zeros

    bufs = ((rows0, cols0, vals0, sem_r0, sem_c0, sem_v0),
            (rows1, cols1, vals1, sem_r1, sem_c1, sem_v1))
    nnz_base = p * NNZ + sl * NNZ_W

    def start_chunk(k):
        r, cl, v, sr, sc_, sv = bufs[k % 2]
        off = nnz_base + k * CHUNK
        return (pltpu.async_copy(rows_hbm.at[pl.ds(off, CHUNK)], r, sr),
                pltpu.async_copy(cols_hbm.at[pl.ds(off, CHUNK)], cl, sc_),
                pltpu.async_copy(vals_hbm.at[pl.ds(off, CHUNK)], v, sv))

    pending = start_chunk(0)
    for k in range(NCHUNKS):
        for d in pending:
            d.wait()
        if k + 1 < NCHUNKS:
            pending = start_chunk(k + 1)
        r, cl, v = bufs[k % 2][:3]

        @plsc.parallel_loop(0, CHUNK // L, unroll=4)
        def _spmm(j):
            rows = r[pl.ds(j * L, L)]
            cols = cl[pl.ds(j * L, L)]
            vals = v[pl.ds(j * L, L)]
            gx = plsc.load_gather(x_v, [cols])
            plsc.addupdate_scatter(ax_v, [rows], vals * gx)
            gl = plsc.load_gather(lam_v, [rows])
            plsc.addupdate_scatter(at_v, [cols], vals * gl)

    pltpu.sync_copy(ax_v, sh_ax.at[s])
    pltpu.sync_copy(at_v, sh_at.at[s])
    plsc.subcore_barrier()

    row0 = (s // W_PER_P) * W_PER_P
    mbase = sl * RED
    for j in range(W_PER_P):
        pltpu.sync_copy(sh_ax.at[row0 + j, pl.ds(mbase, RED)],
                        ax_v.at[pl.ds(j * RED, RED)])
        pltpu.sync_copy(sh_at.at[row0 + j, pl.ds(mbase, RED)],
                        at_v.at[pl.ds(j * RED, RED)])
    pltpu.sync_copy(b_hbm.at[pl.ds(p * M + mbase, RED)], vals0.at[pl.ds(0, RED)])
    pltpu.sync_copy(c_hbm.at[pl.ds(p * N + mbase, RED)], vals0.at[pl.ds(RED, RED)])

    carry0 = (zeros, zeros, zeros, zeros)

    @plsc.parallel_loop(0, RED // L, unroll=2, carry=carry0)
    def _red(i, carry):
        vp, vd, vs_, vc = carry
        ax = ax_v[pl.ds(i * L, L)]
        at = at_v[pl.ds(i * L, L)]
        for j in range(1, W_PER_P):
            ax = ax + ax_v[pl.ds(j * RED + i * L, L)]
            at = at + at_v[pl.ds(j * RED + i * L, L)]
        amb = ax - vals0[pl.ds(i * L, L)]
        lamv = lam_v[pl.ds(mbase + i * L, L)]
        rp = jnp.maximum(amb, 0.0)
        rd = jnp.maximum(-lamv, 0.0)
        st = at + vals0[pl.ds(RED + i * L, L)]
        cm = lamv * amb
        return (vp + rp * rp, vd + rd * rd, vs_ + st * st, vc + cm * cm)

    vp, vd, vs_, vc = _red
    out_v[0] = vp
    out_v[1] = vd
    out_v[2] = vs_
    out_v[3] = vc
    pltpu.sync_copy(out_v, out_hbm.at[c, s])


_kkt_sc = pl.kernel(_kkt_sc_body, **_KERNEL_KWARGS)


def kernel(x_hat, lam_hat, A_rows, A_cols, A_vals, b_pad, c_pad, b_mask, c_mask):
    del b_mask, c_mask
    parts = _kkt_sc(
        x_hat.astype(jnp.float32),
        lam_hat.astype(jnp.float32),
        A_rows.reshape(-1),
        A_cols.reshape(-1),
        A_vals.reshape(-1).astype(jnp.float32),
        b_pad.reshape(-1).astype(jnp.float32),
        c_pad.reshape(-1).astype(jnp.float32),
    )
    sums = jnp.sum(parts, axis=(0, 1, 3))
    inv_B = 1.0 / B
    return (W_PRIMAL * (sums[0] / M) * inv_B
            + W_DUAL * (sums[1] / M) * inv_B
            + W_STAT * (sums[2] / N) * inv_B
            + W_COMP * (sums[3] / M) * inv_B)


# speedup vs baseline: 375.9709x; 1.0285x over previous
import jax
import jax.numpy as jnp
from jax import lax
from jax.experimental import pallas as pl
from jax.experimental.pallas import tpu as pltpu
from jax.experimental.pallas import tpu_sc as plsc

B = 4
M = 16384
N = 16384
NNZ = 262144
W_PRIMAL, W_DUAL, W_STAT, W_COMP = 0.1, 0.1, 0.6, 0.2

NC = 2
NS = 16
L = 16
W_PER_P = 8
NNZ_W = NNZ // W_PER_P
CHUNK = 4096
NCHUNKS = NNZ_W // CHUNK
RED = M // W_PER_P

_mesh = plsc.VectorSubcoreMesh(core_axis_name="c", subcore_axis_name="s",
                               num_cores=NC, num_subcores=NS)

_KERNEL_KWARGS = dict(
    out_type=jax.ShapeDtypeStruct((NC, NS, 4, L), jnp.float32),
    mesh=_mesh,
    compiler_params=pltpu.CompilerParams(needs_layout_passes=False),
    scratch_types=[
        pltpu.VMEM((N,), jnp.float32),
        pltpu.VMEM((M,), jnp.float32),
        pltpu.VMEM((M,), jnp.float32),
        pltpu.VMEM((N,), jnp.float32),
        pltpu.VMEM((CHUNK,), jnp.int32),
        pltpu.VMEM((CHUNK,), jnp.int32),
        pltpu.VMEM((CHUNK,), jnp.float32),
        pltpu.VMEM((CHUNK,), jnp.int32),
        pltpu.VMEM((CHUNK,), jnp.int32),
        pltpu.VMEM((CHUNK,), jnp.float32),
        pltpu.VMEM((4, L), jnp.float32),
        pltpu.VMEM_SHARED((NS, M), jnp.float32),
        pltpu.VMEM_SHARED((NS, N), jnp.float32),
        pltpu.SemaphoreType.DMA,
        pltpu.SemaphoreType.DMA,
        pltpu.SemaphoreType.DMA,
        pltpu.SemaphoreType.DMA,
        pltpu.SemaphoreType.DMA,
        pltpu.SemaphoreType.DMA,
        pltpu.SemaphoreType.DMA,
        pltpu.SemaphoreType.DMA,
    ],
)


def _kkt_sc_body(x_hbm, lam_hbm, rows_hbm, cols_hbm, vals_hbm, b_hbm, c_hbm,
                 out_hbm, x_v, lam_v, ax_v, at_v,
                 rows0, cols0, vals0, rows1, cols1, vals1,
                 out_v, sh_ax, sh_at,
                 sem_r0, sem_c0, sem_v0, sem_r1, sem_c1, sem_v1,
                 sem_x, sem_l):
    c = lax.axis_index("c")
    s = lax.axis_index("s")
    p = c * (B // NC) + s // W_PER_P
    sl = s % W_PER_P

    dx = pltpu.async_copy(x_hbm.at[pl.ds(p * N, N)], x_v, sem_x)
    dl = pltpu.async_copy(lam_hbm.at[pl.ds(p * M, M)], lam_v, sem_l)

    zeros = jnp.zeros((L,), jnp.float32)

    @plsc.parallel_loop(0, M // L, unroll=8)
    def _zero(i):
        ax_v[pl.ds(i * L, L)] = zeros
        at_v[pl.ds(i * L, L)] = zeros

    dx.wait()
    dl.wait()

    bufs = ((rows0, cols0, vals0, sem_r0, sem_c0, sem_v0),
            (rows1, cols1, vals1, sem_r1, sem_c1, sem_v1))
    nnz_base = p * NNZ + sl * NNZ_W

    def start_chunk(k):
        r, cl, v, sr, sc_, sv = bufs[k % 2]
        off = nnz_base + k * CHUNK
        return (pltpu.async_copy(rows_hbm.at[pl.ds(off, CHUNK)], r, sr),
                pltpu.async_copy(cols_hbm.at[pl.ds(off, CHUNK)], cl, sc_),
                pltpu.async_copy(vals_hbm.at[pl.ds(off, CHUNK)], v, sv))

    pending = start_chunk(0)
    for k in range(NCHUNKS):
        for d in pending:
            d.wait()
        if k + 1 < NCHUNKS:
            pending = start_chunk(k + 1)
        r, cl, v = bufs[k % 2][:3]

        @plsc.parallel_loop(0, CHUNK // L, unroll=8)
        def _spmm(j):
            rows = r[pl.ds(j * L, L)]
            cols = cl[pl.ds(j * L, L)]
            vals = v[pl.ds(j * L, L)]
            gx = plsc.load_gather(x_v, [cols])
            plsc.addupdate_scatter(ax_v, [rows], vals * gx)
            gl = plsc.load_gather(lam_v, [rows])
            plsc.addupdate_scatter(at_v, [cols], vals * gl)

    pltpu.sync_copy(ax_v, sh_ax.at[s])
    pltpu.sync_copy(at_v, sh_at.at[s])
    plsc.subcore_barrier()

    row0 = (s // W_PER_P) * W_PER_P
    mbase = sl * RED
    for j in range(W_PER_P):
        pltpu.sync_copy(sh_ax.at[row0 + j, pl.ds(mbase, RED)],
                        ax_v.at[pl.ds(j * RED, RED)])
        pltpu.sync_copy(sh_at.at[row0 + j, pl.ds(mbase, RED)],
                        at_v.at[pl.ds(j * RED, RED)])
    pltpu.sync_copy(b_hbm.at[pl.ds(p * M + mbase, RED)], vals0.at[pl.ds(0, RED)])
    pltpu.sync_copy(c_hbm.at[pl.ds(p * N + mbase, RED)], vals0.at[pl.ds(RED, RED)])

    carry0 = (zeros, zeros, zeros, zeros)

    @plsc.parallel_loop(0, RED // L, unroll=2, carry=carry0)
    def _red(i, carry):
        vp, vd, vs_, vc = carry
        ax = ax_v[pl.ds(i * L, L)]
        at = at_v[pl.ds(i * L, L)]
        for j in range(1, W_PER_P):
            ax = ax + ax_v[pl.ds(j * RED + i * L, L)]
            at = at + at_v[pl.ds(j * RED + i * L, L)]
        amb = ax - vals0[pl.ds(i * L, L)]
        lamv = lam_v[pl.ds(mbase + i * L, L)]
        rp = jnp.maximum(amb, 0.0)
        rd = jnp.maximum(-lamv, 0.0)
        st = at + vals0[pl.ds(RED + i * L, L)]
        cm = lamv * amb
        return (vp + rp * rp, vd + rd * rd, vs_ + st * st, vc + cm * cm)

    vp, vd, vs_, vc = _red
    out_v[0] = vp
    out_v[1] = vd
    out_v[2] = vs_
    out_v[3] = vc
    pltpu.sync_copy(out_v, out_hbm.at[c, s])


_kkt_sc = pl.kernel(_kkt_sc_body, **_KERNEL_KWARGS)


def kernel(x_hat, lam_hat, A_rows, A_cols, A_vals, b_pad, c_pad, b_mask, c_mask):
    del b_mask, c_mask
    parts = _kkt_sc(
        x_hat.astype(jnp.float32),
        lam_hat.astype(jnp.float32),
        A_rows.reshape(-1),
        A_cols.reshape(-1),
        A_vals.reshape(-1).astype(jnp.float32),
        b_pad.reshape(-1).astype(jnp.float32),
        c_pad.reshape(-1).astype(jnp.float32),
    )
    sums = jnp.sum(parts, axis=(0, 1, 3))
    inv_B = 1.0 / B
    return (W_PRIMAL * (sums[0] / M) * inv_B
            + W_DUAL * (sums[1] / M) * inv_B
            + W_STAT * (sums[2] / N) * inv_B
            + W_COMP * (sums[3] / M) * inv_B)


# speedup vs baseline: 447.8709x; 1.1912x over previous
import jax
import jax.numpy as jnp
from jax import lax
from jax.experimental import pallas as pl
from jax.experimental.pallas import tpu as pltpu
from jax.experimental.pallas import tpu_sc as plsc

B = 4
M = 16384
N = 16384
NNZ = 262144
W_PRIMAL, W_DUAL, W_STAT, W_COMP = 0.1, 0.1, 0.6, 0.2

NC = 2
NS = 16
L = 16
W_PER_P = 8
NNZ_W = NNZ // W_PER_P
CHUNK = 4096
NCHUNKS = NNZ_W // CHUNK
RED = M // W_PER_P

_mesh = plsc.VectorSubcoreMesh(core_axis_name="c", subcore_axis_name="s",
                               num_cores=NC, num_subcores=NS)

_KERNEL_KWARGS = dict(
    out_type=jax.ShapeDtypeStruct((NC, NS, 4, L), jnp.float32),
    mesh=_mesh,
    compiler_params=pltpu.CompilerParams(needs_layout_passes=False),
    scratch_types=[
        pltpu.VMEM((N,), jnp.float32),
        pltpu.VMEM((M,), jnp.float32),
        pltpu.VMEM((M,), jnp.float32),
        pltpu.VMEM((N,), jnp.float32),
        pltpu.VMEM((CHUNK,), jnp.int32),
        pltpu.VMEM((CHUNK,), jnp.int32),
        pltpu.VMEM((CHUNK,), jnp.float32),
        pltpu.VMEM((CHUNK,), jnp.int32),
        pltpu.VMEM((CHUNK,), jnp.int32),
        pltpu.VMEM((CHUNK,), jnp.float32),
        pltpu.VMEM((4, L), jnp.float32),
        pltpu.VMEM_SHARED((NS, M), jnp.float32),
        pltpu.VMEM_SHARED((NS, N), jnp.float32),
        pltpu.SemaphoreType.DMA,
        pltpu.SemaphoreType.DMA,
        pltpu.SemaphoreType.DMA,
        pltpu.SemaphoreType.DMA,
        pltpu.SemaphoreType.DMA,
        pltpu.SemaphoreType.DMA,
        pltpu.SemaphoreType.DMA,
        pltpu.SemaphoreType.DMA,
    ],
)


def _kkt_sc_body(x_hbm, lam_hbm, rows_hbm, cols_hbm, vals_hbm, b_hbm, c_hbm,
                 out_hbm, x_v, lam_v, ax_v, at_v,
                 rows0, cols0, vals0, rows1, cols1, vals1,
                 out_v, sh_ax, sh_at,
                 sem_r0, sem_c0, sem_v0, sem_r1, sem_c1, sem_v1,
                 sem_x, sem_l):
    c = lax.axis_index("c")
    s = lax.axis_index("s")
    p = c * (B // NC) + s // W_PER_P
    sl = s % W_PER_P

    dx = pltpu.async_copy(x_hbm.at[pl.ds(p * N, N)], x_v, sem_x)
    dl = pltpu.async_copy(lam_hbm.at[pl.ds(p * M, M)], lam_v, sem_l)

    zeros = jnp.zeros((L,), jnp.float32)

    @plsc.parallel_loop(0, M // L, unroll=8)
    def _zero(i):
        ax_v[pl.ds(i * L, L)] = zeros
        at_v[pl.ds(i * L, L)] = zeros

    dx.wait()
    dl.wait()

    bufs = ((rows0, cols0, vals0, sem_r0, sem_c0, sem_v0),
            (rows1, cols1, vals1, sem_r1, sem_c1, sem_v1))
    nnz_base = sl * NNZ_W

    def start_chunk(k):
        r, cl, v, sr, sc_, sv = bufs[k % 2]
        off = nnz_base + k * CHUNK
        return (pltpu.async_copy(rows_hbm.at[p, pl.ds(off, CHUNK)], r, sr),
                pltpu.async_copy(cols_hbm.at[p, pl.ds(off, CHUNK)], cl, sc_),
                pltpu.async_copy(vals_hbm.at[p, pl.ds(off, CHUNK)], v, sv))

    pending = start_chunk(0)
    for k in range(NCHUNKS):
        for d in pending:
            d.wait()
        if k + 1 < NCHUNKS:
            pending = start_chunk(k + 1)
        r, cl, v = bufs[k % 2][:3]

        @plsc.parallel_loop(0, CHUNK // L, unroll=8)
        def _spmm(j):
            rows = r[pl.ds(j * L, L)]
            cols = cl[pl.ds(j * L, L)]
            vals = v[pl.ds(j * L, L)]
            gx = plsc.load_gather(x_v, [cols])
            plsc.addupdate_scatter(ax_v, [rows], vals * gx)
            gl = plsc.load_gather(lam_v, [rows])
            plsc.addupdate_scatter(at_v, [cols], vals * gl)

    pltpu.sync_copy(ax_v, sh_ax.at[s])
    pltpu.sync_copy(at_v, sh_at.at[s])
    plsc.subcore_barrier()

    row0 = (s // W_PER_P) * W_PER_P
    mbase = sl * RED
    for j in range(W_PER_P):
        pltpu.sync_copy(sh_ax.at[row0 + j, pl.ds(mbase, RED)],
                        ax_v.at[pl.ds(j * RED, RED)])
        pltpu.sync_copy(sh_at.at[row0 + j, pl.ds(mbase, RED)],
                        at_v.at[pl.ds(j * RED, RED)])
    pltpu.sync_copy(b_hbm.at[p, pl.ds(mbase, RED)], vals0.at[pl.ds(0, RED)])
    pltpu.sync_copy(c_hbm.at[p, pl.ds(mbase, RED)], vals0.at[pl.ds(RED, RED)])

    carry0 = (zeros, zeros, zeros, zeros)

    @plsc.parallel_loop(0, RED // L, unroll=2, carry=carry0)
    def _red(i, carry):
        vp, vd, vs_, vc = carry
        ax = ax_v[pl.ds(i * L, L)]
        at = at_v[pl.ds(i * L, L)]
        for j in range(1, W_PER_P):
            ax = ax + ax_v[pl.ds(j * RED + i * L, L)]
            at = at + at_v[pl.ds(j * RED + i * L, L)]
        amb = ax - vals0[pl.ds(i * L, L)]
        lamv = lam_v[pl.ds(mbase + i * L, L)]
        rp = jnp.maximum(amb, 0.0)
        rd = jnp.maximum(-lamv, 0.0)
        st = at + vals0[pl.ds(RED + i * L, L)]
        cm = lamv * amb
        return (vp + rp * rp, vd + rd * rd, vs_ + st * st, vc + cm * cm)

    vp, vd, vs_, vc = _red
    out_v[0] = vp
    out_v[1] = vd
    out_v[2] = vs_
    out_v[3] = vc
    pltpu.sync_copy(out_v, out_hbm.at[c, s])


_kkt_sc = pl.kernel(_kkt_sc_body, **_KERNEL_KWARGS)


def kernel(x_hat, lam_hat, A_rows, A_cols, A_vals, b_pad, c_pad, b_mask, c_mask):
    del b_mask, c_mask
    parts = _kkt_sc(x_hat, lam_hat, A_rows, A_cols, A_vals, b_pad, c_pad)
    sums = jnp.sum(parts, axis=(0, 1, 3))
    inv_B = 1.0 / B
    return (W_PRIMAL * (sums[0] / M) * inv_B
            + W_DUAL * (sums[1] / M) * inv_B
            + W_STAT * (sums[2] / N) * inv_B
            + W_COMP * (sums[3] / M) * inv_B)
